# Initial kernel scaffold; baseline (speedup 1.0000x reference)
import jax
import jax.numpy as jnp
from jax.experimental import pallas as pl


def kernel(x_user, x_item, edge_index_rates, edge_index_rated_by, W1_rates, b1_rates, W1_rb, b1_rb, W2_rates, b2_rates, W2_rb, b2_rb):
    raise NotImplementedError("write your pallas kernel here")


# speedup vs baseline: 5.7799x; 5.7799x over previous
import functools

import jax
import jax.numpy as jnp
from jax import lax
from jax.experimental import pallas as pl
from jax.experimental.pallas import tpu as pltpu
from jax.experimental.pallas import tpu_sc as plsc

N_USER = 10000
N_ITEM = 10000
N = 10000
E = 160000
D_IN = 128
D_HID = 128
D_OUT = 64

NC = 2
NS = 16
NW = NC * NS
B = 125
ROWS_PER_W = E // NW // B
ROWS_PER_S = E // NS // B
NPAD = 10240
ZCH = NPAD // NS


def _mesh():
    return plsc.VectorSubcoreMesh(
        core_axis_name="c", subcore_axis_name="s", num_cores=NC, num_subcores=NS
    )


def _sc_degrees(eidx2d, ones_hbm_in, zeros1):
    @functools.partial(
        pl.kernel,
        out_type=[jax.ShapeDtypeStruct((2 * NPAD,), jnp.float32)] * 2,
        mesh=_mesh(),
        scratch_types=[
            pltpu.VMEM((ROWS_PER_S, B), jnp.int32),
            pltpu.VMEM((ROWS_PER_S, B), jnp.int32),
            pltpu.VMEM((B,), jnp.float32),
            pltpu.VMEM_SHARED((NPAD,), jnp.float32),
            pltpu.VMEM_SHARED((NPAD,), jnp.float32),
        ],
    )
    def k(eidx_hbm, ones_hbm, zeros_hbm, outs_hbm, outd_hbm,
          idxs_v, idxd_v, ones_v, degs_sh, degd_sh):
        c = lax.axis_index("c")
        s = lax.axis_index("s")
        pltpu.sync_copy(zeros_hbm, degs_sh.at[pl.ds(s * ZCH, ZCH)])
        pltpu.sync_copy(zeros_hbm, degd_sh.at[pl.ds(s * ZCH, ZCH)])
        pltpu.sync_copy(ones_hbm, ones_v)
        src_row0 = c * (2 * E // B) + s * ROWS_PER_S
        dst_row0 = c * (2 * E // B) + (E // B) + s * ROWS_PER_S
        pltpu.sync_copy(eidx_hbm.at[pl.ds(src_row0, ROWS_PER_S)], idxs_v)
        pltpu.sync_copy(eidx_hbm.at[pl.ds(dst_row0, ROWS_PER_S)], idxd_v)
        plsc.subcore_barrier()

        def body(j, carry):
            pltpu.sync_copy(ones_v, degs_sh.at[idxs_v.at[j]], add=True)
            pltpu.sync_copy(ones_v, degd_sh.at[idxd_v.at[j]], add=True)
            return carry

        lax.fori_loop(0, ROWS_PER_S, body, 0)
        plsc.subcore_barrier()
        pltpu.sync_copy(degs_sh.at[pl.ds(s * ZCH, ZCH)],
                        outs_hbm.at[pl.ds(c * NPAD + s * ZCH, ZCH)])
        pltpu.sync_copy(degd_sh.at[pl.ds(s * ZCH, ZCH)],
                        outd_hbm.at[pl.ds(c * NPAD + s * ZCH, ZCH)])

    return k(eidx2d, ones_hbm_in, zeros1)


def _sc_aggregate(table, src2d, dst2d, zeros2d, d):
    @functools.partial(
        pl.kernel,
        out_type=jax.ShapeDtypeStruct((2 * NPAD, d), jnp.float32),
        mesh=_mesh(),
        scratch_types=[
            pltpu.VMEM((ROWS_PER_W, B), jnp.int32),
            pltpu.VMEM((ROWS_PER_W, B), jnp.int32),
            pltpu.VMEM((B, d), jnp.float32),
            pltpu.VMEM_SHARED((NPAD, d), jnp.float32),
            pltpu.SemaphoreType.DMA,
        ],
    )
    def k(table_hbm, src_hbm, dst_hbm, zeros_hbm, out_hbm,
          idxs_v, idxd_v, rows_v, acc_sh, sem):
        c = lax.axis_index("c")
        s = lax.axis_index("s")
        w = c * NS + s
        pltpu.sync_copy(zeros_hbm, acc_sh.at[pl.ds(s * ZCH, ZCH)])
        pltpu.sync_copy(src_hbm.at[pl.ds(w * ROWS_PER_W, ROWS_PER_W)], idxs_v)
        pltpu.sync_copy(dst_hbm.at[pl.ds(w * ROWS_PER_W, ROWS_PER_W)], idxd_v)
        plsc.subcore_barrier()

        def body(j, carry):
            pltpu.async_copy(table_hbm.at[idxs_v.at[j]], rows_v, sem).wait()
            pltpu.sync_copy(rows_v, acc_sh.at[idxd_v.at[j]], add=True)
            return carry

        lax.fori_loop(0, ROWS_PER_W, body, 0)
        plsc.subcore_barrier()
        pltpu.sync_copy(acc_sh.at[pl.ds(s * ZCH, ZCH)],
                        out_hbm.at[pl.ds(c * NPAD + s * ZCH, ZCH)])

    return k(table, src2d, dst2d, zeros2d)


_MB = 2000


def _nrm(deg):
    return jnp.where(deg > 0.0, lax.rsqrt(jnp.maximum(deg, 1.0)), 0.0)


def _tc_layer1(x_user, x_item, deg_out_r, deg_out_b, w1r, w1b):
    def body(xu_ref, xi_ref, dor_ref, dob_ref, wr_ref, wb_ref, pu_ref, pi_ref):
        pu_ref[...] = jnp.dot(xu_ref[...] * _nrm(dor_ref[...]), wr_ref[...],
                              preferred_element_type=jnp.float32)
        pi_ref[...] = jnp.dot(xi_ref[...] * _nrm(dob_ref[...]), wb_ref[...],
                              preferred_element_type=jnp.float32)

    return pl.pallas_call(
        body,
        grid=(N // _MB,),
        in_specs=[
            pl.BlockSpec((_MB, D_IN), lambda i: (i, 0)),
            pl.BlockSpec((_MB, D_IN), lambda i: (i, 0)),
            pl.BlockSpec((_MB, 1), lambda i: (i, 0)),
            pl.BlockSpec((_MB, 1), lambda i: (i, 0)),
            pl.BlockSpec((D_IN, D_HID), lambda i: (0, 0)),
            pl.BlockSpec((D_IN, D_HID), lambda i: (0, 0)),
        ],
        out_specs=[pl.BlockSpec((_MB, D_HID), lambda i: (i, 0))] * 2,
        out_shape=[jax.ShapeDtypeStruct((N, D_HID), jnp.float32)] * 2,
    )(x_user, x_item, deg_out_r, deg_out_b, w1r, w1b)


def _tc_mid(agg_i, agg_u, deg_in_r, deg_in_b, deg_out_b, deg_out_r,
            b1r, b1b, w2b, w2r):
    def body(ai_ref, au_ref, dir_ref, dib_ref, dob_ref, dor_ref,
             br_ref, bb_ref, wb_ref, wr_ref, p2i_ref, p2u_ref):
        h_item = jnp.maximum(
            (ai_ref[0] + ai_ref[1]) * _nrm(dir_ref[...]) + br_ref[...], 0.0)
        p2i_ref[...] = jnp.dot(h_item * _nrm(dob_ref[...]), wb_ref[...],
                               preferred_element_type=jnp.float32)
        h_user = jnp.maximum(
            (au_ref[0] + au_ref[1]) * _nrm(dib_ref[...]) + bb_ref[...], 0.0)
        p2u_ref[...] = jnp.dot(h_user * _nrm(dor_ref[...]), wr_ref[...],
                               preferred_element_type=jnp.float32)

    return pl.pallas_call(
        body,
        grid=(N // _MB,),
        in_specs=[
            pl.BlockSpec((2, _MB, D_HID), lambda i: (0, i, 0)),
            pl.BlockSpec((2, _MB, D_HID), lambda i: (0, i, 0)),
            pl.BlockSpec((_MB, 1), lambda i: (i, 0)),
            pl.BlockSpec((_MB, 1), lambda i: (i, 0)),
            pl.BlockSpec((_MB, 1), lambda i: (i, 0)),
            pl.BlockSpec((_MB, 1), lambda i: (i, 0)),
            pl.BlockSpec((1, D_HID), lambda i: (0, 0)),
            pl.BlockSpec((1, D_HID), lambda i: (0, 0)),
            pl.BlockSpec((D_HID, D_HID), lambda i: (0, 0)),
            pl.BlockSpec((D_HID, D_HID), lambda i: (0, 0)),
        ],
        out_specs=[pl.BlockSpec((_MB, D_HID), lambda i: (i, 0))] * 2,
        out_shape=[jax.ShapeDtypeStruct((N, D_HID), jnp.float32)] * 2,
    )(agg_i, agg_u, deg_in_r, deg_in_b, deg_out_b, deg_out_r, b1r, b1b, w2b, w2r)


def _tc_final(agg2_i, agg2_u, deg_in_r, deg_in_b, b2r, b2b):
    def body(ai_ref, au_ref, dir_ref, dib_ref, br_ref, bb_ref, oi_ref, ou_ref):
        ai = ai_ref[0, :, :D_OUT] + ai_ref[1, :, :D_OUT]
        au = au_ref[0, :, :D_OUT] + au_ref[1, :, :D_OUT]
        oi_ref[...] = ai * _nrm(dir_ref[...]) + br_ref[...]
        ou_ref[...] = au * _nrm(dib_ref[...]) + bb_ref[...]

    return pl.pallas_call(
        body,
        grid=(N // _MB,),
        in_specs=[
            pl.BlockSpec((2, _MB, D_HID), lambda i: (0, i, 0)),
            pl.BlockSpec((2, _MB, D_HID), lambda i: (0, i, 0)),
            pl.BlockSpec((_MB, 1), lambda i: (i, 0)),
            pl.BlockSpec((_MB, 1), lambda i: (i, 0)),
            pl.BlockSpec((1, D_OUT), lambda i: (0, 0)),
            pl.BlockSpec((1, D_OUT), lambda i: (0, 0)),
        ],
        out_specs=[pl.BlockSpec((_MB, D_OUT), lambda i: (i, 0))] * 2,
        out_shape=[jax.ShapeDtypeStruct((N, D_OUT), jnp.float32)] * 2,
    )(agg2_i, agg2_u, deg_in_r, deg_in_b, b2r, b2b)


def kernel(x_user, x_item, edge_index_rates, edge_index_rated_by,
           W1_rates, b1_rates, W1_rb, b1_rb,
           W2_rates, b2_rates, W2_rb, b2_rb):
    src_r = edge_index_rates[0].astype(jnp.int32)
    dst_r = edge_index_rates[1].astype(jnp.int32)
    src_b = edge_index_rated_by[0].astype(jnp.int32)
    dst_b = edge_index_rated_by[1].astype(jnp.int32)

    src2d_r = src_r.reshape(E // B, B)
    dst2d_r = dst_r.reshape(E // B, B)
    src2d_b = src_b.reshape(E // B, B)
    dst2d_b = dst_b.reshape(E // B, B)
    eidx2d = jnp.concatenate([src2d_r, dst2d_r, src2d_b, dst2d_b], axis=0)

    zeros1 = jnp.zeros((ZCH,), jnp.float32)
    zeros_hid = jnp.zeros((ZCH, D_HID), jnp.float32)
    ones_b = jnp.ones((B,), jnp.float32)

    deg_src, deg_dst = _sc_degrees(eidx2d, ones_b, zeros1)
    deg_out_r = deg_src[:NPAD].reshape(NPAD, 1)
    deg_in_r = deg_dst[:NPAD].reshape(NPAD, 1)
    deg_out_b = deg_src[NPAD:].reshape(NPAD, 1)
    deg_in_b = deg_dst[NPAD:].reshape(NPAD, 1)

    p1u, p1i = _tc_layer1(x_user, x_item, deg_out_r, deg_out_b,
                          W1_rates, W1_rb)

    agg1_item = _sc_aggregate(p1u, src2d_r, dst2d_r, zeros_hid, D_HID)
    agg1_user = _sc_aggregate(p1i, src2d_b, dst2d_b, zeros_hid, D_HID)
    agg1_item = agg1_item.reshape(2, NPAD, D_HID)
    agg1_user = agg1_user.reshape(2, NPAD, D_HID)

    b1r = b1_rates.reshape(1, D_HID)
    b1b = b1_rb.reshape(1, D_HID)
    w2b_p = jnp.pad(W2_rb, ((0, 0), (0, D_HID - D_OUT)))
    w2r_p = jnp.pad(W2_rates, ((0, 0), (0, D_HID - D_OUT)))
    p2i, p2u = _tc_mid(agg1_item, agg1_user, deg_in_r, deg_in_b,
                       deg_out_b, deg_out_r, b1r, b1b, w2b_p, w2r_p)

    agg2_item = _sc_aggregate(p2u, src2d_r, dst2d_r, zeros_hid, D_HID)
    agg2_user = _sc_aggregate(p2i, src2d_b, dst2d_b, zeros_hid, D_HID)
    agg2_item = agg2_item.reshape(2, NPAD, D_HID)
    agg2_user = agg2_user.reshape(2, NPAD, D_HID)

    out_item, out_user = _tc_final(agg2_item, agg2_user, deg_in_r, deg_in_b,
                                   b2_rates.reshape(1, D_OUT),
                                   b2_rb.reshape(1, D_OUT))
    return (out_user, out_item)


# speedup vs baseline: 8.4095x; 1.4550x over previous
import functools

import jax
import jax.numpy as jnp
from jax import lax
from jax.experimental import pallas as pl
from jax.experimental.pallas import tpu as pltpu
from jax.experimental.pallas import tpu_sc as plsc

N_USER = 10000
N_ITEM = 10000
N = 10000
E = 160000
D_IN = 128
D_HID = 128
D_OUT = 64

NC = 2
NS = 16
NW = NC * NS
B = 125
ROWS_PER_W = E // NW // B
ROWS_PER_S = E // NS // B
NPAD = 10240
ZCH = NPAD // NS


def _mesh():
    return plsc.VectorSubcoreMesh(
        core_axis_name="c", subcore_axis_name="s", num_cores=NC, num_subcores=NS
    )


def _sc_degrees(eidx2d, ones_hbm_in, zeros1):
    @functools.partial(
        pl.kernel,
        out_type=[jax.ShapeDtypeStruct((2 * NPAD,), jnp.float32)] * 2,
        mesh=_mesh(),
        scratch_types=[
            pltpu.VMEM((ROWS_PER_S, B), jnp.int32),
            pltpu.VMEM((ROWS_PER_S, B), jnp.int32),
            pltpu.VMEM((B,), jnp.float32),
            pltpu.VMEM_SHARED((NPAD,), jnp.float32),
            pltpu.VMEM_SHARED((NPAD,), jnp.float32),
        ],
    )
    def k(eidx_hbm, ones_hbm, zeros_hbm, outs_hbm, outd_hbm,
          idxs_v, idxd_v, ones_v, degs_sh, degd_sh):
        c = lax.axis_index("c")
        s = lax.axis_index("s")
        pltpu.sync_copy(zeros_hbm, degs_sh.at[pl.ds(s * ZCH, ZCH)])
        pltpu.sync_copy(zeros_hbm, degd_sh.at[pl.ds(s * ZCH, ZCH)])
        pltpu.sync_copy(ones_hbm, ones_v)
        src_row0 = c * (2 * E // B) + s * ROWS_PER_S
        dst_row0 = c * (2 * E // B) + (E // B) + s * ROWS_PER_S
        pltpu.sync_copy(eidx_hbm.at[pl.ds(src_row0, ROWS_PER_S)], idxs_v)
        pltpu.sync_copy(eidx_hbm.at[pl.ds(dst_row0, ROWS_PER_S)], idxd_v)
        plsc.subcore_barrier()

        def body(j, carry):
            pltpu.sync_copy(ones_v, degs_sh.at[idxs_v.at[j]], add=True)
            pltpu.sync_copy(ones_v, degd_sh.at[idxd_v.at[j]], add=True)
            return carry

        lax.fori_loop(0, ROWS_PER_S, body, 0)
        plsc.subcore_barrier()
        pltpu.sync_copy(degs_sh.at[pl.ds(s * ZCH, ZCH)],
                        outs_hbm.at[pl.ds(c * NPAD + s * ZCH, ZCH)])
        pltpu.sync_copy(degd_sh.at[pl.ds(s * ZCH, ZCH)],
                        outd_hbm.at[pl.ds(c * NPAD + s * ZCH, ZCH)])

    return k(eidx2d, ones_hbm_in, zeros1)


ROWS_PER_C = E // NS // B
_CH = 16


def _sc_aggregate(table2, src2d, dst2d, zeros2d, d):
    @functools.partial(
        pl.kernel,
        out_type=jax.ShapeDtypeStruct((2 * NPAD, d), jnp.float32),
        mesh=_mesh(),
        scratch_types=[
            pltpu.VMEM((_CH, B), jnp.int32),
            pltpu.VMEM((_CH, B), jnp.int32),
            pltpu.VMEM((2, B, d), jnp.float32),
            pltpu.VMEM_SHARED((NPAD, d), jnp.float32),
            pltpu.SemaphoreType.DMA,
            pltpu.SemaphoreType.DMA,
            pltpu.SemaphoreType.DMA,
            pltpu.SemaphoreType.DMA,
        ],
    )
    def k(table_hbm, src_hbm, dst_hbm, zeros_hbm, out_hbm,
          idxs_v, idxd_v, rows_v, acc_sh, gs0, gs1, ss0, ss1):
        gsem = (gs0, gs1)
        ssem = (ss0, ss1)
        c = lax.axis_index("c")
        s = lax.axis_index("s")
        row0 = c * (E // B) + s * ROWS_PER_C
        pltpu.sync_copy(zeros_hbm, acc_sh.at[pl.ds(s * ZCH, ZCH)])
        plsc.subcore_barrier()

        def chunk(ci, carry):
            r0 = row0 + ci * _CH
            pltpu.sync_copy(src_hbm.at[pl.ds(r0, _CH)], idxs_v)
            pltpu.sync_copy(dst_hbm.at[pl.ds(r0, _CH)], idxd_v)
            for b in range(2):
                pltpu.async_copy(
                    table_hbm.at[idxs_v.at[b]], rows_v.at[b], gsem[b])

            def pair(g, carry2):
                for b in range(2):
                    jj = g * 2 + b
                    pltpu.make_async_copy(
                        table_hbm.at[idxs_v.at[jj]], rows_v.at[b],
                        gsem[b]).wait()
                    sd = pltpu.async_copy(
                        rows_v.at[b], acc_sh.at[idxd_v.at[jj]], ssem[b],
                        add=True)
                    sd.wait()
                    pltpu.async_copy(
                        table_hbm.at[idxs_v.at[jj + 2]], rows_v.at[b], gsem[b])
                return carry2

            lax.fori_loop(0, _CH // 2 - 1, pair, 0)

            for b in range(2):
                jj = _CH - 2 + b
                pltpu.make_async_copy(
                    table_hbm.at[idxs_v.at[jj]], rows_v.at[b], gsem[b]).wait()
                pltpu.async_copy(
                    rows_v.at[b], acc_sh.at[idxd_v.at[jj]], ssem[b],
                    add=True).wait()
            return carry

        lax.fori_loop(0, ROWS_PER_C // _CH, chunk, 0)

        plsc.subcore_barrier()
        pltpu.sync_copy(acc_sh.at[pl.ds(s * ZCH, ZCH)],
                        out_hbm.at[pl.ds(c * NPAD + s * ZCH, ZCH)])

    return k(table2, src2d, dst2d, zeros2d)


_MB = 2000


def _nrm(deg):
    return jnp.where(deg > 0.0, lax.rsqrt(jnp.maximum(deg, 1.0)), 0.0)


def _tc_layer1(x_user, x_item, deg_out_r, deg_out_b, w1r, w1b):
    def body(xu_ref, xi_ref, dor_ref, dob_ref, wr_ref, wb_ref, t_ref):
        t_ref[0] = jnp.dot(xu_ref[...] * _nrm(dor_ref[...]), wr_ref[...],
                           preferred_element_type=jnp.float32)
        t_ref[1] = jnp.dot(xi_ref[...] * _nrm(dob_ref[...]), wb_ref[...],
                           preferred_element_type=jnp.float32)

    return pl.pallas_call(
        body,
        grid=(N // _MB,),
        in_specs=[
            pl.BlockSpec((_MB, D_IN), lambda i: (i, 0)),
            pl.BlockSpec((_MB, D_IN), lambda i: (i, 0)),
            pl.BlockSpec((_MB, 1), lambda i: (i, 0)),
            pl.BlockSpec((_MB, 1), lambda i: (i, 0)),
            pl.BlockSpec((D_IN, D_HID), lambda i: (0, 0)),
            pl.BlockSpec((D_IN, D_HID), lambda i: (0, 0)),
        ],
        out_specs=pl.BlockSpec((2, _MB, D_HID), lambda i: (0, i, 0)),
        out_shape=jax.ShapeDtypeStruct((2, N, D_HID), jnp.float32),
    )(x_user, x_item, deg_out_r, deg_out_b, w1r, w1b)


def _tc_mid(agg1, deg_in_r, deg_in_b, deg_out_b, deg_out_r, b1r, b1b, w2b, w2r):
    def body(a_ref, dir_ref, dib_ref, dob_ref, dor_ref,
             br_ref, bb_ref, wb_ref, wr_ref, t_ref):
        h_item = jnp.maximum(
            a_ref[0] * _nrm(dir_ref[...]) + br_ref[...], 0.0)
        h_user = jnp.maximum(
            a_ref[1] * _nrm(dib_ref[...]) + bb_ref[...], 0.0)
        t_ref[0] = jnp.dot(h_user * _nrm(dor_ref[...]), wr_ref[...],
                           preferred_element_type=jnp.float32)
        t_ref[1] = jnp.dot(h_item * _nrm(dob_ref[...]), wb_ref[...],
                           preferred_element_type=jnp.float32)

    return pl.pallas_call(
        body,
        grid=(N // _MB,),
        in_specs=[
            pl.BlockSpec((2, _MB, D_HID), lambda i: (0, i, 0)),
            pl.BlockSpec((_MB, 1), lambda i: (i, 0)),
            pl.BlockSpec((_MB, 1), lambda i: (i, 0)),
            pl.BlockSpec((_MB, 1), lambda i: (i, 0)),
            pl.BlockSpec((_MB, 1), lambda i: (i, 0)),
            pl.BlockSpec((1, D_HID), lambda i: (0, 0)),
            pl.BlockSpec((1, D_HID), lambda i: (0, 0)),
            pl.BlockSpec((D_HID, D_HID), lambda i: (0, 0)),
            pl.BlockSpec((D_HID, D_HID), lambda i: (0, 0)),
        ],
        out_specs=pl.BlockSpec((2, _MB, D_HID), lambda i: (0, i, 0)),
        out_shape=jax.ShapeDtypeStruct((2, N, D_HID), jnp.float32),
    )(agg1, deg_in_r, deg_in_b, deg_out_b, deg_out_r, b1r, b1b, w2b, w2r)


def _tc_final(agg2, deg_in_r, deg_in_b, b2r, b2b):
    def body(a_ref, dir_ref, dib_ref, br_ref, bb_ref, oi_ref, ou_ref):
        oi_ref[...] = a_ref[0, :, :D_OUT] * _nrm(dir_ref[...]) + br_ref[...]
        ou_ref[...] = a_ref[1, :, :D_OUT] * _nrm(dib_ref[...]) + bb_ref[...]

    return pl.pallas_call(
        body,
        grid=(N // _MB,),
        in_specs=[
            pl.BlockSpec((2, _MB, D_HID), lambda i: (0, i, 0)),
            pl.BlockSpec((_MB, 1), lambda i: (i, 0)),
            pl.BlockSpec((_MB, 1), lambda i: (i, 0)),
            pl.BlockSpec((1, D_OUT), lambda i: (0, 0)),
            pl.BlockSpec((1, D_OUT), lambda i: (0, 0)),
        ],
        out_specs=[pl.BlockSpec((_MB, D_OUT), lambda i: (i, 0))] * 2,
        out_shape=[jax.ShapeDtypeStruct((N, D_OUT), jnp.float32)] * 2,
    )(agg2, deg_in_r, deg_in_b, b2r, b2b)


def kernel(x_user, x_item, edge_index_rates, edge_index_rated_by,
           W1_rates, b1_rates, W1_rb, b1_rb,
           W2_rates, b2_rates, W2_rb, b2_rb):
    src_r = edge_index_rates[0].astype(jnp.int32)
    dst_r = edge_index_rates[1].astype(jnp.int32)
    src_b = edge_index_rated_by[0].astype(jnp.int32)
    dst_b = edge_index_rated_by[1].astype(jnp.int32)

    src2d = jnp.concatenate([src_r, src_b + N]).reshape(2 * E // B, B)
    dst2d = jnp.concatenate([dst_r, dst_b]).reshape(2 * E // B, B)
    eidx2d = jnp.concatenate([src_r, dst_r, src_b, dst_b]).reshape(
        4 * E // B, B)

    zeros1 = jnp.zeros((ZCH,), jnp.float32)
    zeros_hid = jnp.zeros((ZCH, D_HID), jnp.float32)
    ones_b = jnp.ones((B,), jnp.float32)

    deg_src, deg_dst = _sc_degrees(eidx2d, ones_b, zeros1)
    deg_out_r = deg_src[:NPAD].reshape(NPAD, 1)
    deg_in_r = deg_dst[:NPAD].reshape(NPAD, 1)
    deg_out_b = deg_src[NPAD:].reshape(NPAD, 1)
    deg_in_b = deg_dst[NPAD:].reshape(NPAD, 1)

    t1 = _tc_layer1(x_user, x_item, deg_out_r, deg_out_b,
                    W1_rates, W1_rb).reshape(2 * N, D_HID)

    agg1 = _sc_aggregate(t1, src2d, dst2d, zeros_hid, D_HID)
    agg1 = agg1.reshape(2, NPAD, D_HID)

    b1r = b1_rates.reshape(1, D_HID)
    b1b = b1_rb.reshape(1, D_HID)
    w2b_p = jnp.pad(W2_rb, ((0, 0), (0, D_HID - D_OUT)))
    w2r_p = jnp.pad(W2_rates, ((0, 0), (0, D_HID - D_OUT)))
    t2 = _tc_mid(agg1, deg_in_r, deg_in_b, deg_out_b, deg_out_r,
                 b1r, b1b, w2b_p, w2r_p).reshape(2 * N, D_HID)

    agg2 = _sc_aggregate(t2, src2d, dst2d, zeros_hid, D_HID)
    agg2 = agg2.reshape(2, NPAD, D_HID)

    out_item, out_user = _tc_final(agg2, deg_in_r, deg_in_b,
                                   b2_rates.reshape(1, D_OUT),
                                   b2_rb.reshape(1, D_OUT))
    return (out_user, out_item)


# speedup vs baseline: 8.8232x; 1.0492x over previous
import functools

import jax
import jax.numpy as jnp
from jax import lax
from jax.experimental import pallas as pl
from jax.experimental.pallas import tpu as pltpu
from jax.experimental.pallas import tpu_sc as plsc

N_USER = 10000
N_ITEM = 10000
N = 10000
E = 160000
D_IN = 128
D_HID = 128
D_OUT = 64

NC = 2
NS = 16
NW = NC * NS
B = 125
ROWS_PER_W = E // NW // B
ROWS_PER_S = E // NS // B
NPAD = 10240
ZCH = NPAD // NS


def _mesh():
    return plsc.VectorSubcoreMesh(
        core_axis_name="c", subcore_axis_name="s", num_cores=NC, num_subcores=NS
    )


def _sc_degrees(eidx2d, ones_hbm_in, zeros1):
    @functools.partial(
        pl.kernel,
        out_type=[jax.ShapeDtypeStruct((2 * NPAD,), jnp.float32)] * 2,
        mesh=_mesh(),
        scratch_types=[
            pltpu.VMEM((ROWS_PER_S, B), jnp.int32),
            pltpu.VMEM((ROWS_PER_S, B), jnp.int32),
            pltpu.VMEM((B,), jnp.float32),
            pltpu.VMEM_SHARED((NPAD,), jnp.float32),
            pltpu.VMEM_SHARED((NPAD,), jnp.float32),
        ],
    )
    def k(eidx_hbm, ones_hbm, zeros_hbm, outs_hbm, outd_hbm,
          idxs_v, idxd_v, ones_v, degs_sh, degd_sh):
        c = lax.axis_index("c")
        s = lax.axis_index("s")
        pltpu.sync_copy(zeros_hbm, degs_sh.at[pl.ds(s * ZCH, ZCH)])
        pltpu.sync_copy(zeros_hbm, degd_sh.at[pl.ds(s * ZCH, ZCH)])
        pltpu.sync_copy(ones_hbm, ones_v)
        src_row0 = c * (2 * E // B) + s * ROWS_PER_S
        dst_row0 = c * (2 * E // B) + (E // B) + s * ROWS_PER_S
        pltpu.sync_copy(eidx_hbm.at[pl.ds(src_row0, ROWS_PER_S)], idxs_v)
        pltpu.sync_copy(eidx_hbm.at[pl.ds(dst_row0, ROWS_PER_S)], idxd_v)
        plsc.subcore_barrier()

        def body(j, carry):
            pltpu.sync_copy(ones_v, degs_sh.at[idxs_v.at[j]], add=True)
            pltpu.sync_copy(ones_v, degd_sh.at[idxd_v.at[j]], add=True)
            return carry

        lax.fori_loop(0, ROWS_PER_S, body, 0)
        plsc.subcore_barrier()
        pltpu.sync_copy(degs_sh.at[pl.ds(s * ZCH, ZCH)],
                        outs_hbm.at[pl.ds(c * NPAD + s * ZCH, ZCH)])
        pltpu.sync_copy(degd_sh.at[pl.ds(s * ZCH, ZCH)],
                        outd_hbm.at[pl.ds(c * NPAD + s * ZCH, ZCH)])

    return k(eidx2d, ones_hbm_in, zeros1)


ROWS_PER_C = E // NS // B
_CH = 16


def _sc_aggregate(table2, src2d, dst2d, zeros2d, d):
    @functools.partial(
        pl.kernel,
        out_type=jax.ShapeDtypeStruct((2 * NPAD, d), jnp.float32),
        mesh=_mesh(),
        scratch_types=[
            pltpu.VMEM((2, _CH, B), jnp.int32),
            pltpu.VMEM((2, _CH, B), jnp.int32),
            pltpu.VMEM((2, B, d), jnp.float32),
            pltpu.VMEM_SHARED((NPAD, d), jnp.float32),
            pltpu.SemaphoreType.DMA,
            pltpu.SemaphoreType.DMA,
            pltpu.SemaphoreType.DMA,
            pltpu.SemaphoreType.DMA,
            pltpu.SemaphoreType.DMA,
        ],
    )
    def k(table_hbm, src_hbm, dst_hbm, zeros_hbm, out_hbm,
          idxs_v, idxd_v, rows_v, acc_sh, gs0, gs1, ss0, ss1, isem):
        gsem = (gs0, gs1)
        ssem = (ss0, ss1)
        c = lax.axis_index("c")
        s = lax.axis_index("s")
        row0 = c * (E // B) + s * ROWS_PER_C
        pltpu.sync_copy(zeros_hbm, acc_sh.at[pl.ds(s * ZCH, ZCH)])
        plsc.subcore_barrier()

        n_chunks = ROWS_PER_C // _CH

        def _idx_load(ci, ib, sync):
            r0 = row0 + ci * _CH
            if sync:
                pltpu.sync_copy(src_hbm.at[pl.ds(r0, _CH)], idxs_v.at[ib])
                pltpu.sync_copy(dst_hbm.at[pl.ds(r0, _CH)], idxd_v.at[ib])
            else:
                pltpu.async_copy(src_hbm.at[pl.ds(r0, _CH)], idxs_v.at[ib],
                                 isem)
                pltpu.async_copy(dst_hbm.at[pl.ds(r0, _CH)], idxd_v.at[ib],
                                 isem)

        def _idx_wait(ci, ib):
            r0 = row0 + ci * _CH
            pltpu.make_async_copy(src_hbm.at[pl.ds(r0, _CH)], idxs_v.at[ib],
                                  isem).wait()
            pltpu.make_async_copy(dst_hbm.at[pl.ds(r0, _CH)], idxd_v.at[ib],
                                  isem).wait()

        def _step(ib, jj, b, refill_ib, refill_jj):
            pltpu.make_async_copy(
                table_hbm.at[idxs_v.at[ib, jj]], rows_v.at[b], gsem[b]).wait()
            sd = pltpu.async_copy(
                rows_v.at[b], acc_sh.at[idxd_v.at[ib, jj]], ssem[b], add=True)
            sd.wait()
            if refill_ib is not None:
                pltpu.async_copy(
                    table_hbm.at[idxs_v.at[refill_ib, refill_jj]],
                    rows_v.at[b], gsem[b])

        _idx_load(0, 0, sync=True)
        _idx_load(1, 1, sync=False)
        for b in range(2):
            pltpu.async_copy(table_hbm.at[idxs_v.at[0, b]], rows_v.at[b],
                             gsem[b])

        for ci in range(n_chunks):
            ib = ci % 2
            nib = (ci + 1) % 2
            last = ci == n_chunks - 1

            def pair(g, carry, ib=ib):
                for b in range(2):
                    jj = g * 2 + b
                    _step(ib, jj, b, ib, jj + 2)
                return carry

            lax.fori_loop(0, _CH // 2 - 1, pair, 0)

            if not last:
                _idx_wait(ci + 1, nib)
                _step(ib, _CH - 2, 0, nib, 0)
                _step(ib, _CH - 1, 1, nib, 1)
                if ci + 2 < n_chunks:
                    _idx_load(ci + 2, ib, sync=False)
            else:
                _step(ib, _CH - 2, 0, None, None)
                _step(ib, _CH - 1, 1, None, None)

        plsc.subcore_barrier()
        pltpu.sync_copy(acc_sh.at[pl.ds(s * ZCH, ZCH)],
                        out_hbm.at[pl.ds(c * NPAD + s * ZCH, ZCH)])

    return k(table2, src2d, dst2d, zeros2d)


_MB = 2000


def _nrm(deg):
    return jnp.where(deg > 0.0, lax.rsqrt(jnp.maximum(deg, 1.0)), 0.0)


def _tc_layer1(x_user, x_item, deg_out_r, deg_out_b, w1r, w1b):
    def body(xu_ref, xi_ref, dor_ref, dob_ref, wr_ref, wb_ref, t_ref):
        t_ref[0] = jnp.dot(xu_ref[...] * _nrm(dor_ref[...]), wr_ref[...],
                           preferred_element_type=jnp.float32)
        t_ref[1] = jnp.dot(xi_ref[...] * _nrm(dob_ref[...]), wb_ref[...],
                           preferred_element_type=jnp.float32)

    return pl.pallas_call(
        body,
        grid=(N // _MB,),
        in_specs=[
            pl.BlockSpec((_MB, D_IN), lambda i: (i, 0)),
            pl.BlockSpec((_MB, D_IN), lambda i: (i, 0)),
            pl.BlockSpec((_MB, 1), lambda i: (i, 0)),
            pl.BlockSpec((_MB, 1), lambda i: (i, 0)),
            pl.BlockSpec((D_IN, D_HID), lambda i: (0, 0)),
            pl.BlockSpec((D_IN, D_HID), lambda i: (0, 0)),
        ],
        out_specs=pl.BlockSpec((2, _MB, D_HID), lambda i: (0, i, 0)),
        out_shape=jax.ShapeDtypeStruct((2, N, D_HID), jnp.float32),
    )(x_user, x_item, deg_out_r, deg_out_b, w1r, w1b)


def _tc_mid(agg1, deg_in_r, deg_in_b, deg_out_b, deg_out_r, b1r, b1b, w2b, w2r):
    def body(a_ref, dir_ref, dib_ref, dob_ref, dor_ref,
             br_ref, bb_ref, wb_ref, wr_ref, t_ref):
        h_item = jnp.maximum(
            a_ref[0] * _nrm(dir_ref[...]) + br_ref[...], 0.0)
        h_user = jnp.maximum(
            a_ref[1] * _nrm(dib_ref[...]) + bb_ref[...], 0.0)
        t_ref[0] = jnp.dot(h_user * _nrm(dor_ref[...]), wr_ref[...],
                           preferred_element_type=jnp.float32)
        t_ref[1] = jnp.dot(h_item * _nrm(dob_ref[...]), wb_ref[...],
                           preferred_element_type=jnp.float32)

    return pl.pallas_call(
        body,
        grid=(N // _MB,),
        in_specs=[
            pl.BlockSpec((2, _MB, D_HID), lambda i: (0, i, 0)),
            pl.BlockSpec((_MB, 1), lambda i: (i, 0)),
            pl.BlockSpec((_MB, 1), lambda i: (i, 0)),
            pl.BlockSpec((_MB, 1), lambda i: (i, 0)),
            pl.BlockSpec((_MB, 1), lambda i: (i, 0)),
            pl.BlockSpec((1, D_HID), lambda i: (0, 0)),
            pl.BlockSpec((1, D_HID), lambda i: (0, 0)),
            pl.BlockSpec((D_HID, D_HID), lambda i: (0, 0)),
            pl.BlockSpec((D_HID, D_HID), lambda i: (0, 0)),
        ],
        out_specs=pl.BlockSpec((2, _MB, D_HID), lambda i: (0, i, 0)),
        out_shape=jax.ShapeDtypeStruct((2, N, D_HID), jnp.float32),
    )(agg1, deg_in_r, deg_in_b, deg_out_b, deg_out_r, b1r, b1b, w2b, w2r)


def _tc_final(agg2, deg_in_r, deg_in_b, b2r, b2b):
    def body(a_ref, dir_ref, dib_ref, br_ref, bb_ref, oi_ref, ou_ref):
        oi_ref[...] = a_ref[0, :, :D_OUT] * _nrm(dir_ref[...]) + br_ref[...]
        ou_ref[...] = a_ref[1, :, :D_OUT] * _nrm(dib_ref[...]) + bb_ref[...]

    return pl.pallas_call(
        body,
        grid=(N // _MB,),
        in_specs=[
            pl.BlockSpec((2, _MB, D_HID), lambda i: (0, i, 0)),
            pl.BlockSpec((_MB, 1), lambda i: (i, 0)),
            pl.BlockSpec((_MB, 1), lambda i: (i, 0)),
            pl.BlockSpec((1, D_OUT), lambda i: (0, 0)),
            pl.BlockSpec((1, D_OUT), lambda i: (0, 0)),
        ],
        out_specs=[pl.BlockSpec((_MB, D_OUT), lambda i: (i, 0))] * 2,
        out_shape=[jax.ShapeDtypeStruct((N, D_OUT), jnp.float32)] * 2,
    )(agg2, deg_in_r, deg_in_b, b2r, b2b)


def kernel(x_user, x_item, edge_index_rates, edge_index_rated_by,
           W1_rates, b1_rates, W1_rb, b1_rb,
           W2_rates, b2_rates, W2_rb, b2_rb):
    src_r = edge_index_rates[0].astype(jnp.int32)
    dst_r = edge_index_rates[1].astype(jnp.int32)
    src_b = edge_index_rated_by[0].astype(jnp.int32)
    dst_b = edge_index_rated_by[1].astype(jnp.int32)

    src2d = jnp.concatenate([src_r, src_b + N]).reshape(2 * E // B, B)
    dst2d = jnp.concatenate([dst_r, dst_b]).reshape(2 * E // B, B)
    eidx2d = jnp.concatenate([src_r, dst_r, src_b, dst_b]).reshape(
        4 * E // B, B)

    zeros1 = jnp.zeros((ZCH,), jnp.float32)
    zeros_hid = jnp.zeros((ZCH, D_HID), jnp.float32)
    ones_b = jnp.ones((B,), jnp.float32)

    deg_src, deg_dst = _sc_degrees(eidx2d, ones_b, zeros1)
    deg_out_r = deg_src[:NPAD].reshape(NPAD, 1)
    deg_in_r = deg_dst[:NPAD].reshape(NPAD, 1)
    deg_out_b = deg_src[NPAD:].reshape(NPAD, 1)
    deg_in_b = deg_dst[NPAD:].reshape(NPAD, 1)

    t1 = _tc_layer1(x_user, x_item, deg_out_r, deg_out_b,
                    W1_rates, W1_rb).reshape(2 * N, D_HID)

    agg1 = _sc_aggregate(t1, src2d, dst2d, zeros_hid, D_HID)
    agg1 = agg1.reshape(2, NPAD, D_HID)

    b1r = b1_rates.reshape(1, D_HID)
    b1b = b1_rb.reshape(1, D_HID)
    w2b_p = jnp.pad(W2_rb, ((0, 0), (0, D_HID - D_OUT)))
    w2r_p = jnp.pad(W2_rates, ((0, 0), (0, D_HID - D_OUT)))
    t2 = _tc_mid(agg1, deg_in_r, deg_in_b, deg_out_b, deg_out_r,
                 b1r, b1b, w2b_p, w2r_p).reshape(2 * N, D_HID)

    agg2 = _sc_aggregate(t2, src2d, dst2d, zeros_hid, D_HID)
    agg2 = agg2.reshape(2, NPAD, D_HID)

    out_item, out_user = _tc_final(agg2, deg_in_r, deg_in_b,
                                   b2_rates.reshape(1, D_OUT),
                                   b2_rb.reshape(1, D_OUT))
    return (out_user, out_item)


# speedup vs baseline: 9.0368x; 1.0242x over previous
import functools

import jax
import jax.numpy as jnp
from jax import lax
from jax.experimental import pallas as pl
from jax.experimental.pallas import tpu as pltpu
from jax.experimental.pallas import tpu_sc as plsc

N_USER = 10000
N_ITEM = 10000
N = 10000
E = 160000
D_IN = 128
D_HID = 128
D_OUT = 64

NC = 2
NS = 16
NW = NC * NS
B = 125
ROWS_PER_W = E // NW // B
ROWS_PER_S = E // NS // B
NPAD = 10240
ZCH = NPAD // NS


def _mesh():
    return plsc.VectorSubcoreMesh(
        core_axis_name="c", subcore_axis_name="s", num_cores=NC, num_subcores=NS
    )


def _sc_degrees(eidx2d, ones_hbm_in, zeros1):
    @functools.partial(
        pl.kernel,
        out_type=[jax.ShapeDtypeStruct((2 * NPAD,), jnp.float32)] * 2,
        mesh=_mesh(),
        scratch_types=[
            pltpu.VMEM((ROWS_PER_S, B), jnp.int32),
            pltpu.VMEM((ROWS_PER_S, B), jnp.int32),
            pltpu.VMEM((B,), jnp.float32),
            pltpu.VMEM_SHARED((NPAD,), jnp.float32),
            pltpu.VMEM_SHARED((NPAD,), jnp.float32),
        ],
    )
    def k(eidx_hbm, ones_hbm, zeros_hbm, outs_hbm, outd_hbm,
          idxs_v, idxd_v, ones_v, degs_sh, degd_sh):
        c = lax.axis_index("c")
        s = lax.axis_index("s")
        pltpu.sync_copy(zeros_hbm, degs_sh.at[pl.ds(s * ZCH, ZCH)])
        pltpu.sync_copy(zeros_hbm, degd_sh.at[pl.ds(s * ZCH, ZCH)])
        pltpu.sync_copy(ones_hbm, ones_v)
        src_row0 = c * (2 * E // B) + s * ROWS_PER_S
        dst_row0 = c * (2 * E // B) + (E // B) + s * ROWS_PER_S
        pltpu.sync_copy(eidx_hbm.at[pl.ds(src_row0, ROWS_PER_S)], idxs_v)
        pltpu.sync_copy(eidx_hbm.at[pl.ds(dst_row0, ROWS_PER_S)], idxd_v)
        plsc.subcore_barrier()

        def body(j, carry):
            pltpu.sync_copy(ones_v, degs_sh.at[idxs_v.at[j]], add=True)
            pltpu.sync_copy(ones_v, degd_sh.at[idxd_v.at[j]], add=True)
            return carry

        lax.fori_loop(0, ROWS_PER_S, body, 0)
        plsc.subcore_barrier()
        pltpu.sync_copy(degs_sh.at[pl.ds(s * ZCH, ZCH)],
                        outs_hbm.at[pl.ds(c * NPAD + s * ZCH, ZCH)])
        pltpu.sync_copy(degd_sh.at[pl.ds(s * ZCH, ZCH)],
                        outd_hbm.at[pl.ds(c * NPAD + s * ZCH, ZCH)])

    return k(eidx2d, ones_hbm_in, zeros1)


ROWS_PER_C = E // NS // B
_CH = 16


def _sc_aggregate(table_r, table_b, er2, eb2, zeros2d, d):
    @functools.partial(
        pl.kernel,
        out_type=jax.ShapeDtypeStruct((2 * NPAD, d), jnp.float32),
        mesh=_mesh(),
        scratch_types=[
            pltpu.VMEM((2, _CH, B), jnp.int32),
            pltpu.VMEM((2, _CH, B), jnp.int32),
            pltpu.VMEM((2, B, d), jnp.float32),
            pltpu.VMEM_SHARED((NPAD, d), jnp.float32),
            pltpu.SemaphoreType.DMA,
            pltpu.SemaphoreType.DMA,
            pltpu.SemaphoreType.DMA,
            pltpu.SemaphoreType.DMA,
            pltpu.SemaphoreType.DMA,
        ],
    )
    def k(tr_hbm, tb_hbm, er_hbm, eb_hbm, zeros_hbm, out_hbm,
          idxs_v, idxd_v, rows_v, acc_sh, gs0, gs1, ss0, ss1, isem):
        gsem = (gs0, gs1)
        ssem = (ss0, ss1)
        c = lax.axis_index("c")
        s = lax.axis_index("s")
        row0 = s * ROWS_PER_C
        pltpu.sync_copy(zeros_hbm, acc_sh.at[pl.ds(s * ZCH, ZCH)])
        plsc.subcore_barrier()

        n_chunks = ROWS_PER_C // _CH

        def _idx_load(ci, ib, sync):
            r0 = row0 + ci * _CH

            @pl.when(c == 0)
            def _():
                if sync:
                    pltpu.sync_copy(er_hbm.at[pl.ds(r0, _CH)], idxs_v.at[ib])
                    pltpu.sync_copy(er_hbm.at[pl.ds(E // B + r0, _CH)],
                                    idxd_v.at[ib])
                else:
                    pltpu.async_copy(er_hbm.at[pl.ds(r0, _CH)],
                                     idxs_v.at[ib], isem)
                    pltpu.async_copy(er_hbm.at[pl.ds(E // B + r0, _CH)],
                                     idxd_v.at[ib], isem)

            @pl.when(c == 1)
            def _():
                if sync:
                    pltpu.sync_copy(eb_hbm.at[pl.ds(r0, _CH)], idxs_v.at[ib])
                    pltpu.sync_copy(eb_hbm.at[pl.ds(E // B + r0, _CH)],
                                    idxd_v.at[ib])
                else:
                    pltpu.async_copy(eb_hbm.at[pl.ds(r0, _CH)],
                                     idxs_v.at[ib], isem)
                    pltpu.async_copy(eb_hbm.at[pl.ds(E // B + r0, _CH)],
                                     idxd_v.at[ib], isem)

        def _idx_wait(ci, ib):
            r0 = row0 + ci * _CH
            pltpu.make_async_copy(er_hbm.at[pl.ds(r0, _CH)],
                                  idxs_v.at[ib], isem).wait()
            pltpu.make_async_copy(er_hbm.at[pl.ds(E // B + r0, _CH)],
                                  idxd_v.at[ib], isem).wait()

        def _gather_start(ib, jj, b):
            @pl.when(c == 0)
            def _():
                pltpu.async_copy(tr_hbm.at[idxs_v.at[ib, jj]], rows_v.at[b],
                                 gsem[b])

            @pl.when(c == 1)
            def _():
                pltpu.async_copy(tb_hbm.at[idxs_v.at[ib, jj]], rows_v.at[b],
                                 gsem[b])

        def _step(ib, jj, b, refill_ib, refill_jj):
            pltpu.make_async_copy(
                tr_hbm.at[idxs_v.at[ib, jj]], rows_v.at[b], gsem[b]).wait()
            sd = pltpu.async_copy(
                rows_v.at[b], acc_sh.at[idxd_v.at[ib, jj]], ssem[b], add=True)
            sd.wait()
            if refill_ib is not None:
                _gather_start(refill_ib, refill_jj, b)

        _idx_load(0, 0, sync=True)
        _idx_load(1, 1, sync=False)
        for b in range(2):
            _gather_start(0, b, b)

        for ci in range(n_chunks):
            ib = ci % 2
            nib = (ci + 1) % 2
            last = ci == n_chunks - 1

            def pair(g, carry, ib=ib):
                for b in range(2):
                    jj = g * 2 + b
                    _step(ib, jj, b, ib, jj + 2)
                return carry

            lax.fori_loop(0, _CH // 2 - 1, pair, 0)

            if not last:
                _idx_wait(ci + 1, nib)
                _step(ib, _CH - 2, 0, nib, 0)
                _step(ib, _CH - 1, 1, nib, 1)
                if ci + 2 < n_chunks:
                    _idx_load(ci + 2, ib, sync=False)
            else:
                _step(ib, _CH - 2, 0, None, None)
                _step(ib, _CH - 1, 1, None, None)

        plsc.subcore_barrier()
        pltpu.sync_copy(acc_sh.at[pl.ds(s * ZCH, ZCH)],
                        out_hbm.at[pl.ds(c * NPAD + s * ZCH, ZCH)])

    return k(table_r, table_b, er2, eb2, zeros2d)


_MB = 2000


def _nrm(deg):
    return jnp.where(deg > 0.0, lax.rsqrt(jnp.maximum(deg, 1.0)), 0.0)


def _nrm_blk(deg_ref, i):
    del i
    return _nrm(deg_ref[...])


_DEG_SPEC = pl.BlockSpec((_MB, 1), lambda i: (i, 0))


def _tc_layer1(x_user, x_item, deg_out_r, deg_out_b, w1r, w1b):
    def body(xu_ref, xi_ref, dor_ref, dob_ref, wr_ref, wb_ref,
             tr_ref, tb_ref):
        i = pl.program_id(0)
        tr_ref[...] = jnp.dot(xu_ref[...] * _nrm_blk(dor_ref, i), wr_ref[...],
                              preferred_element_type=jnp.float32)
        tb_ref[...] = jnp.dot(xi_ref[...] * _nrm_blk(dob_ref, i), wb_ref[...],
                              preferred_element_type=jnp.float32)

    return pl.pallas_call(
        body,
        grid=(N // _MB,),
        in_specs=[
            pl.BlockSpec((_MB, D_IN), lambda i: (i, 0)),
            pl.BlockSpec((_MB, D_IN), lambda i: (i, 0)),
            _DEG_SPEC,
            _DEG_SPEC,
            pl.BlockSpec((D_IN, D_HID), lambda i: (0, 0)),
            pl.BlockSpec((D_IN, D_HID), lambda i: (0, 0)),
        ],
        out_specs=[pl.BlockSpec((_MB, D_HID), lambda i: (i, 0))] * 2,
        out_shape=[jax.ShapeDtypeStruct((N, D_HID), jnp.float32)] * 2,
    )(x_user, x_item, deg_out_r, deg_out_b, w1r, w1b)


def _tc_mid(agg1, deg_in_r, deg_in_b, deg_out_b, deg_out_r, b1r, b1b,
            w2b, w2r):
    def body(a_ref, dir_ref, dib_ref, dob_ref, dor_ref,
             br_ref, bb_ref, wb_ref, wr_ref, tr_ref, tb_ref):
        i = pl.program_id(0)
        h_item = jnp.maximum(
            a_ref[0] * _nrm_blk(dir_ref, i) + br_ref[...], 0.0)
        h_user = jnp.maximum(
            a_ref[1] * _nrm_blk(dib_ref, i) + bb_ref[...], 0.0)
        tr_ref[...] = jnp.dot(h_user * _nrm_blk(dor_ref, i), wr_ref[...],
                              preferred_element_type=jnp.float32)
        tb_ref[...] = jnp.dot(h_item * _nrm_blk(dob_ref, i), wb_ref[...],
                              preferred_element_type=jnp.float32)

    return pl.pallas_call(
        body,
        grid=(N // _MB,),
        in_specs=[
            pl.BlockSpec((2, _MB, D_HID), lambda i: (0, i, 0)),
            _DEG_SPEC,
            _DEG_SPEC,
            _DEG_SPEC,
            _DEG_SPEC,
            pl.BlockSpec((1, D_HID), lambda i: (0, 0)),
            pl.BlockSpec((1, D_HID), lambda i: (0, 0)),
            pl.BlockSpec((D_HID, D_HID), lambda i: (0, 0)),
            pl.BlockSpec((D_HID, D_HID), lambda i: (0, 0)),
        ],
        out_specs=[pl.BlockSpec((_MB, D_HID), lambda i: (i, 0))] * 2,
        out_shape=[jax.ShapeDtypeStruct((N, D_HID), jnp.float32)] * 2,
    )(agg1, deg_in_r, deg_in_b, deg_out_b, deg_out_r, b1r, b1b, w2b, w2r)


def _tc_final(agg2, deg_in_r, deg_in_b, b2r, b2b):
    def body(a_ref, dir_ref, dib_ref, br_ref, bb_ref, oi_ref, ou_ref):
        i = pl.program_id(0)
        oi_ref[...] = (a_ref[0, :, :D_OUT] * _nrm_blk(dir_ref, i)
                       + br_ref[...])
        ou_ref[...] = (a_ref[1, :, :D_OUT] * _nrm_blk(dib_ref, i)
                       + bb_ref[...])

    return pl.pallas_call(
        body,
        grid=(N // _MB,),
        in_specs=[
            pl.BlockSpec((2, _MB, D_HID), lambda i: (0, i, 0)),
            _DEG_SPEC,
            _DEG_SPEC,
            pl.BlockSpec((1, D_OUT), lambda i: (0, 0)),
            pl.BlockSpec((1, D_OUT), lambda i: (0, 0)),
        ],
        out_specs=[pl.BlockSpec((_MB, D_OUT), lambda i: (i, 0))] * 2,
        out_shape=[jax.ShapeDtypeStruct((N, D_OUT), jnp.float32)] * 2,
    )(agg2, deg_in_r, deg_in_b, b2r, b2b)


def kernel(x_user, x_item, edge_index_rates, edge_index_rated_by,
           W1_rates, b1_rates, W1_rb, b1_rb,
           W2_rates, b2_rates, W2_rb, b2_rb):
    er2 = edge_index_rates.astype(jnp.int32).reshape(2 * E // B, B)
    eb2 = edge_index_rated_by.astype(jnp.int32).reshape(2 * E // B, B)

    zeros1 = jnp.zeros((ZCH,), jnp.float32)
    zeros_hid = jnp.zeros((ZCH, D_HID), jnp.float32)
    ones_b = jnp.ones((B,), jnp.float32)

    eidx2d = jnp.concatenate([er2, eb2], axis=0)
    deg_src, deg_dst = _sc_degrees(eidx2d, ones_b, zeros1)
    deg_out_r = deg_src[:NPAD].reshape(NPAD, 1)
    deg_in_r = deg_dst[:NPAD].reshape(NPAD, 1)
    deg_out_b = deg_src[NPAD:].reshape(NPAD, 1)
    deg_in_b = deg_dst[NPAD:].reshape(NPAD, 1)

    t1r, t1b = _tc_layer1(x_user, x_item, deg_out_r, deg_out_b,
                          W1_rates, W1_rb)

    agg1 = _sc_aggregate(t1r, t1b, er2, eb2, zeros_hid, D_HID)
    agg1 = agg1.reshape(2, NPAD, D_HID)

    b1r = b1_rates.reshape(1, D_HID)
    b1b = b1_rb.reshape(1, D_HID)
    w2b_p = jnp.pad(W2_rb, ((0, 0), (0, D_HID - D_OUT)))
    w2r_p = jnp.pad(W2_rates, ((0, 0), (0, D_HID - D_OUT)))
    t2r, t2b = _tc_mid(agg1, deg_in_r, deg_in_b, deg_out_b, deg_out_r,
                       b1r, b1b, w2b_p, w2r_p)

    agg2 = _sc_aggregate(t2r, t2b, er2, eb2, zeros_hid, D_HID)
    agg2 = agg2.reshape(2, NPAD, D_HID)

    out_item, out_user = _tc_final(agg2, deg_in_r, deg_in_b,
                                   b2_rates.reshape(1, D_OUT),
                                   b2_rb.reshape(1, D_OUT))
    return (out_user, out_item)
